# Initial kernel scaffold; baseline (speedup 1.0000x reference)
import jax
import jax.numpy as jnp
from jax import lax
from jax.experimental import pallas as pl
from jax.experimental.pallas import tpu as pltpu
from jax.experimental.pallas import tpu_sc as plsc

_ITERATIONS = 16
_B = 256
_W = 8192
_LANES = 16
_HALO = 8
_BUF = _HALO + _W + _HALO
_NUM_WORKERS = 32
_ROWS_PER_WORKER = _B // _NUM_WORKERS
_CHUNKS = _W // _LANES


def _body(in_hbm, lut_hbm, out_hbm, lut_v, in_v, buf_a, buf_b):
    wid = lax.axis_index("s") * 2 + lax.axis_index("c")

    pltpu.sync_copy(lut_hbm, lut_v)

    zeros = jnp.zeros((_LANES,), jnp.float32)
    for buf in (buf_a, buf_b):
        buf[pl.ds(0, _LANES)] = zeros
        buf[pl.ds(_BUF - _LANES, _LANES)] = zeros

    def step(src, dst):
        def chunk(i, carry):
            base = i * _LANES
            l = src[pl.ds(_HALO - 1 + base, _LANES)]
            c = src[pl.ds(_HALO + base, _LANES)]
            r = src[pl.ds(_HALO + 1 + base, _LANES)]
            idx = (l + c * 2.0 + r * 4.0).astype(jnp.int32)
            dst[pl.ds(_HALO + base, _LANES)] = plsc.load_gather(lut_v, [idx])
            return carry

        lax.fori_loop(0, _CHUNKS, chunk, 0)

    def row_body(rr, carry):
        b = wid * _ROWS_PER_WORKER + rr
        pltpu.sync_copy(in_hbm.at[b], in_v)

        def thresh(i, c2):
            base = i * _LANES
            v = in_v[pl.ds(base, _LANES)]
            buf_a[pl.ds(_HALO + base, _LANES)] = jnp.where(v >= 0.5, 1.0, 0.0)
            return c2

        lax.fori_loop(0, _CHUNKS, thresh, 0)
        pltpu.sync_copy(buf_a.at[pl.ds(_HALO, _W)], out_hbm.at[b, 0])

        src, dst = buf_a, buf_b
        for it in range(_ITERATIONS):
            step(src, dst)
            pltpu.sync_copy(dst.at[pl.ds(_HALO, _W)], out_hbm.at[b, it + 1])
            src, dst = dst, src
        return carry

    lax.fori_loop(0, _ROWS_PER_WORKER, row_body, 0)


@jax.jit
def _run(x, lut16):
    mesh = plsc.VectorSubcoreMesh(core_axis_name="c", subcore_axis_name="s")
    return pl.kernel(
        _body,
        out_type=jax.ShapeDtypeStruct((_B, _ITERATIONS + 1, _W), jnp.float32),
        mesh=mesh,
        scratch_types=[
            pltpu.VMEM((_LANES,), jnp.float32),
            pltpu.VMEM((_W,), jnp.float32),
            pltpu.VMEM((_BUF,), jnp.float32),
            pltpu.VMEM((_BUF,), jnp.float32),
        ],
    )(x, lut16)


def kernel(input, lookup):
    lut16 = jnp.concatenate([lookup, jnp.zeros((8,), jnp.float32)])
    return _run(input, lut16)


# speedup vs baseline: 5.0041x; 5.0041x over previous
import jax
import jax.numpy as jnp
from jax import lax
from jax.experimental import pallas as pl
from jax.experimental.pallas import tpu as pltpu
from jax.experimental.pallas import tpu_sc as plsc

_ITERATIONS = 16
_B = 256
_W = 8192
_LANES = 16
_HALO = 8
_BUF = _HALO + _W + _HALO
_NUM_WORKERS = 32
_ROWS_PER_WORKER = _B // _NUM_WORKERS
_CHUNKS = _W // _LANES


def _body(in_hbm, lut_hbm, out_hbm, lut_v, in_v, buf_a, buf_b):
    wid = lax.axis_index("s") * 2 + lax.axis_index("c")

    pltpu.sync_copy(lut_hbm, lut_v)

    zeros = jnp.zeros((_LANES,), jnp.float32)
    for buf in (buf_a, buf_b):
        buf[pl.ds(0, _LANES)] = zeros
        buf[pl.ds(_BUF - _LANES, _LANES)] = zeros

    tbl = lut_v[...]

    def step(src, dst):
        def chunk(i, carry):
            base = i * _LANES
            l = src[pl.ds(_HALO - 1 + base, _LANES)]
            c = src[pl.ds(_HALO + base, _LANES)]
            r = src[pl.ds(_HALO + 1 + base, _LANES)]
            idx = (l + c * 2.0 + r * 4.0).astype(jnp.int32)
            dst[pl.ds(_HALO + base, _LANES)] = tbl.at[idx].get(
                mode="promise_in_bounds")
            return carry

        lax.fori_loop(0, _CHUNKS, chunk, 0)

    def row_body(rr, carry):
        b = wid * _ROWS_PER_WORKER + rr
        pltpu.sync_copy(in_hbm.at[b], in_v)

        def thresh(i, c2):
            base = i * _LANES
            v = in_v[pl.ds(base, _LANES)]
            buf_a[pl.ds(_HALO + base, _LANES)] = jnp.where(v >= 0.5, 1.0, 0.0)
            return c2

        lax.fori_loop(0, _CHUNKS, thresh, 0)
        orow = b * (_ITERATIONS + 1)
        pltpu.sync_copy(buf_a.at[pl.ds(_HALO, _W)], out_hbm.at[orow])

        src, dst = buf_a, buf_b
        for it in range(_ITERATIONS):
            step(src, dst)
            pltpu.sync_copy(dst.at[pl.ds(_HALO, _W)], out_hbm.at[orow + it + 1])
            src, dst = dst, src
        return carry

    lax.fori_loop(0, _ROWS_PER_WORKER, row_body, 0)


@jax.jit
def _run(x, lut16):
    mesh = plsc.VectorSubcoreMesh(core_axis_name="c", subcore_axis_name="s")
    return pl.kernel(
        _body,
        out_type=jax.ShapeDtypeStruct((_B * (_ITERATIONS + 1), _W), jnp.float32),
        mesh=mesh,
        scratch_types=[
            pltpu.VMEM((_LANES,), jnp.float32),
            pltpu.VMEM((_W,), jnp.float32),
            pltpu.VMEM((_BUF,), jnp.float32),
            pltpu.VMEM((_BUF,), jnp.float32),
        ],
        compiler_params=pltpu.CompilerParams(use_tc_tiling_on_sc=False),
    )(x, lut16)


def kernel(input, lookup):
    lut16 = jnp.concatenate([lookup, jnp.zeros((8,), jnp.float32)])
    out = _run(input, lut16)
    return out.reshape(_B, _ITERATIONS + 1, _W)


# speedup vs baseline: 6.8723x; 1.3733x over previous
import jax
import jax.numpy as jnp
from jax import lax
from jax.experimental import pallas as pl
from jax.experimental.pallas import tpu as pltpu
from jax.experimental.pallas import tpu_sc as plsc

_ITERATIONS = 16
_B = 256
_W = 8192
_LANES = 16
_HALO = 8
_BUF = _HALO + _W + _HALO
_NUM_WORKERS = 32
_ROWS_PER_WORKER = _B // _NUM_WORKERS
_CHUNKS = _W // _LANES


def _body(in_hbm, lut_hbm, out_hbm, lut_v, in_v, buf_a, buf_b, sem_a, sem_b):
    wid = lax.axis_index("s") * 2 + lax.axis_index("c")

    pltpu.sync_copy(lut_hbm, lut_v)
    tbl = lut_v[...]

    zeros = jnp.zeros((_LANES,), jnp.float32)
    for buf in (buf_a, buf_b):
        buf[pl.ds(0, _LANES)] = zeros
        buf[pl.ds(_BUF - _LANES, _LANES)] = zeros

    orow0 = wid * _ROWS_PER_WORKER * (_ITERATIONS + 1)
    pltpu.async_copy(buf_a.at[pl.ds(_HALO, _W)], out_hbm.at[orow0], sem_a)
    pltpu.async_copy(buf_b.at[pl.ds(_HALO, _W)], out_hbm.at[orow0 + 1], sem_b)

    def wait_row_dma(buf, sem, orow):
        pltpu.make_async_copy(
            buf.at[pl.ds(_HALO, _W)], out_hbm.at[orow], sem).wait()

    def step(src, dst):
        @plsc.parallel_loop(0, _CHUNKS, unroll=8)
        def chunk(i):
            base = i * _LANES
            l = src[pl.ds(_HALO - 1 + base, _LANES)]
            c = src[pl.ds(_HALO + base, _LANES)]
            r = src[pl.ds(_HALO + 1 + base, _LANES)]
            idx = (l + c * 2.0 + r * 4.0).astype(jnp.int32)
            dst[pl.ds(_HALO + base, _LANES)] = tbl.at[idx].get(
                mode="promise_in_bounds")

    def row_body(rr, carry):
        b = wid * _ROWS_PER_WORKER + rr
        orow = b * (_ITERATIONS + 1)
        pltpu.sync_copy(in_hbm.at[pl.ds(b * _W, _W)], in_v)

        wait_row_dma(buf_a, sem_a, orow)

        @plsc.parallel_loop(0, _CHUNKS, unroll=8)
        def thresh(i):
            base = i * _LANES
            v = in_v[pl.ds(base, _LANES)]
            buf_a[pl.ds(_HALO + base, _LANES)] = jnp.where(v >= 0.5, 1.0, 0.0)

        pltpu.async_copy(buf_a.at[pl.ds(_HALO, _W)], out_hbm.at[orow], sem_a)

        src, dst = buf_a, buf_b
        sems = {id(buf_a): sem_a, id(buf_b): sem_b}
        for it in range(_ITERATIONS):
            wait_row_dma(dst, sems[id(dst)], orow + it + 1)
            step(src, dst)
            pltpu.async_copy(
                dst.at[pl.ds(_HALO, _W)], out_hbm.at[orow + it + 1],
                sems[id(dst)])
            src, dst = dst, src
        return carry

    lax.fori_loop(0, _ROWS_PER_WORKER, row_body, 0)

    last = (wid + 1) * _ROWS_PER_WORKER * (_ITERATIONS + 1)
    wait_row_dma(buf_a, sem_a, last - 2)
    wait_row_dma(buf_b, sem_b, last - 1)


@jax.jit
def _run(x_flat, lut16):
    mesh = plsc.VectorSubcoreMesh(core_axis_name="c", subcore_axis_name="s")
    return pl.kernel(
        _body,
        out_type=jax.ShapeDtypeStruct((_B * (_ITERATIONS + 1), _W), jnp.float32),
        mesh=mesh,
        scratch_types=[
            pltpu.VMEM((_LANES,), jnp.float32),
            pltpu.VMEM((_W,), jnp.float32),
            pltpu.VMEM((_BUF,), jnp.float32),
            pltpu.VMEM((_BUF,), jnp.float32),
            pltpu.SemaphoreType.DMA,
            pltpu.SemaphoreType.DMA,
        ],
        compiler_params=pltpu.CompilerParams(use_tc_tiling_on_sc=False),
    )(x_flat, lut16)


def kernel(input, lookup):
    lut16 = jnp.concatenate([lookup, jnp.zeros((8,), jnp.float32)])
    out = _run(input.reshape(-1), lut16)
    return out.reshape(_B, _ITERATIONS + 1, _W)


# speedup vs baseline: 9.3365x; 1.3586x over previous
import jax
import jax.numpy as jnp
from jax import lax
from jax.experimental import pallas as pl
from jax.experimental.pallas import tpu as pltpu
from jax.experimental.pallas import tpu_sc as plsc

_ITERATIONS = 16
_B = 256
_W = 8192
_LANES = 16
_HALO = 8
_BUF = _HALO + _W + _HALO
_NUM_WORKERS = 32
_ROWS_PER_WORKER = _B // _NUM_WORKERS
_CHUNKS = _W // _LANES


def _body(in_hbm, lut_hbm, out_hbm, lut_v, buf_a, buf_b, stg,
          sem_lo, sem_hi, sem_q, sem_in):
    wid = lax.axis_index("s") * 2 + lax.axis_index("c")

    pltpu.sync_copy(lut_hbm, lut_v)
    tbl = lut_v[...]

    zeros = jnp.zeros((_LANES,), jnp.float32)
    for buf in (buf_a, buf_b):
        buf[pl.ds(0, _LANES)] = zeros
        buf[pl.ds(_BUF - _LANES, _LANES)] = zeros

    def fire_half(p, b, q, sem):
        pltpu.async_copy(
            stg.at[pl.ds(p, 4), :], out_hbm.at[b, pl.ds(q, 4), :], sem)

    def drain_half(b, q, sem):
        pltpu.make_async_copy(
            stg.at[pl.ds(0, 4), :], out_hbm.at[b, pl.ds(q, 4), :], sem).wait()

    def fire_last(b, sem):
        pltpu.async_copy(
            stg.at[pl.ds(0, 1), :], out_hbm.at[b, pl.ds(16, 1), :], sem)

    def drain_last(b, sem):
        pltpu.make_async_copy(
            stg.at[pl.ds(0, 1), :], out_hbm.at[b, pl.ds(16, 1), :], sem).wait()

    b0 = wid * _ROWS_PER_WORKER
    fire_half(4, b0, 12, sem_hi)
    fire_last(b0, sem_q)

    def row_body(rr, carry):
        b = wid * _ROWS_PER_WORKER + rr
        pltpu.async_copy(
            in_hbm.at[pl.ds(b * _W, _W)], buf_b.at[pl.ds(_HALO, _W)],
            sem_in).wait()

        drain_last(b, sem_q)

        @plsc.parallel_loop(0, _CHUNKS, unroll=8)
        def thresh(i):
            base = i * _LANES
            v = buf_b[pl.ds(_HALO + base, _LANES)]
            s = jnp.where(v >= 0.5, 1.0, 0.0)
            buf_a[pl.ds(_HALO + base, _LANES)] = s
            stg[0, pl.ds(base, _LANES)] = s

        src, dst = buf_a, buf_b
        for k in range(1, _ITERATIONS + 1):
            if k == 4:
                drain_half(b, 12, sem_hi)
            elif k == 8:
                drain_half(b, 0, sem_lo)
            elif k == 12:
                drain_half(b, 4, sem_hi)
            elif k == 16:
                drain_half(b, 8, sem_lo)

            itm = k % 8

            @plsc.parallel_loop(0, _CHUNKS, unroll=8)
            def chunk(i):
                base = i * _LANES
                l = src[pl.ds(_HALO - 1 + base, _LANES)]
                c = src[pl.ds(_HALO + base, _LANES)]
                r = src[pl.ds(_HALO + 1 + base, _LANES)]
                idx = (l + c * 2.0 + r * 4.0).astype(jnp.int32)
                val = tbl.at[idx].get(mode="promise_in_bounds")
                dst[pl.ds(_HALO + base, _LANES)] = val
                stg[itm, pl.ds(base, _LANES)] = val

            if k == 3:
                fire_half(0, b, 0, sem_lo)
            elif k == 7:
                fire_half(4, b, 4, sem_hi)
            elif k == 11:
                fire_half(0, b, 8, sem_lo)
            elif k == 15:
                fire_half(4, b, 12, sem_hi)
            elif k == 16:
                fire_last(b, sem_q)
            src, dst = dst, src
        return carry

    lax.fori_loop(0, _ROWS_PER_WORKER, row_body, 0)

    blast = wid * _ROWS_PER_WORKER + _ROWS_PER_WORKER - 1
    drain_half(blast, 12, sem_hi)
    drain_last(blast, sem_q)


@jax.jit
def _run(x_flat, lut16):
    mesh = plsc.VectorSubcoreMesh(core_axis_name="c", subcore_axis_name="s")
    return pl.kernel(
        _body,
        out_type=jax.ShapeDtypeStruct((_B, _ITERATIONS + 1, _W), jnp.float32),
        mesh=mesh,
        scratch_types=[
            pltpu.VMEM((_LANES,), jnp.float32),
            pltpu.VMEM((_BUF,), jnp.float32),
            pltpu.VMEM((_BUF,), jnp.float32),
            pltpu.VMEM((8, _W), jnp.float32),
            pltpu.SemaphoreType.DMA,
            pltpu.SemaphoreType.DMA,
            pltpu.SemaphoreType.DMA,
            pltpu.SemaphoreType.DMA,
        ],
    )(x_flat, lut16)


def kernel(input, lookup):
    lut16 = jnp.concatenate([lookup, jnp.zeros((8,), jnp.float32)])
    return _run(input.reshape(-1), lut16)


# speedup vs baseline: 9.6409x; 1.0326x over previous
import jax
import jax.numpy as jnp
from jax import lax
from jax.experimental import pallas as pl
from jax.experimental.pallas import tpu as pltpu
from jax.experimental.pallas import tpu_sc as plsc

_ITERATIONS = 16
_B = 256
_W = 8192
_LANES = 16
_HALO = 8
_BUF = _HALO + _W + _HALO
_NUM_WORKERS = 32
_ROWS_PER_WORKER = _B // _NUM_WORKERS
_CHUNKS = _W // _LANES


def _body(in_hbm, lut_hbm, out_hbm, lut_v, in_v, buf_a, buf_b, stg,
          sem_lo, sem_hi, sem_q, sem_in):
    wid = lax.axis_index("s") * 2 + lax.axis_index("c")

    pltpu.sync_copy(lut_hbm, lut_v)
    tbl = lut_v[...]

    zeros = jnp.zeros((_LANES,), jnp.float32)
    for buf in (buf_a, buf_b):
        buf[pl.ds(0, _LANES)] = zeros
        buf[pl.ds(_BUF - _LANES, _LANES)] = zeros

    def fire_half(p, b, q, sem):
        pltpu.async_copy(
            stg.at[pl.ds(p, 4), :], out_hbm.at[b, pl.ds(q, 4), :], sem)

    def drain_half(b, q, sem):
        pltpu.make_async_copy(
            stg.at[pl.ds(0, 4), :], out_hbm.at[b, pl.ds(q, 4), :], sem).wait()

    def fire_last(b, sem):
        pltpu.async_copy(
            stg.at[pl.ds(0, 1), :], out_hbm.at[b, pl.ds(16, 1), :], sem)

    def drain_last(b, sem):
        pltpu.make_async_copy(
            stg.at[pl.ds(0, 1), :], out_hbm.at[b, pl.ds(16, 1), :], sem).wait()

    b0 = wid * _ROWS_PER_WORKER
    fire_half(4, b0, 12, sem_hi)
    fire_last(b0, sem_q)

    def row_body(rr, carry):
        b = wid * _ROWS_PER_WORKER + rr
        pltpu.async_copy(
            in_hbm.at[pl.ds(b, 1), :], in_v, sem_in).wait()

        drain_last(b, sem_q)

        @plsc.parallel_loop(0, _CHUNKS, unroll=8)
        def thresh(i):
            base = i * _LANES
            v = in_v[0, pl.ds(base, _LANES)]
            s = jnp.where(v >= 0.5, 1.0, 0.0)
            buf_a[pl.ds(_HALO + base, _LANES)] = s
            stg[0, pl.ds(base, _LANES)] = s

        src, dst = buf_a, buf_b
        for k in range(1, _ITERATIONS + 1):
            if k == 4:
                drain_half(b, 12, sem_hi)
            elif k == 8:
                drain_half(b, 0, sem_lo)
            elif k == 12:
                drain_half(b, 4, sem_hi)
            elif k == 16:
                drain_half(b, 8, sem_lo)

            itm = k % 8

            @plsc.parallel_loop(0, _CHUNKS, unroll=8)
            def chunk(i):
                base = i * _LANES
                l = src[pl.ds(_HALO - 1 + base, _LANES)]
                c = src[pl.ds(_HALO + base, _LANES)]
                r = src[pl.ds(_HALO + 1 + base, _LANES)]
                idx = (l + c * 2.0 + r * 4.0).astype(jnp.int32)
                val = tbl.at[idx].get(mode="promise_in_bounds")
                dst[pl.ds(_HALO + base, _LANES)] = val
                stg[itm, pl.ds(base, _LANES)] = val

            if k == 3:
                fire_half(0, b, 0, sem_lo)
            elif k == 7:
                fire_half(4, b, 4, sem_hi)
            elif k == 11:
                fire_half(0, b, 8, sem_lo)
            elif k == 15:
                fire_half(4, b, 12, sem_hi)
            elif k == 16:
                fire_last(b, sem_q)
            src, dst = dst, src
        return carry

    lax.fori_loop(0, _ROWS_PER_WORKER, row_body, 0)

    blast = wid * _ROWS_PER_WORKER + _ROWS_PER_WORKER - 1
    drain_half(blast, 12, sem_hi)
    drain_last(blast, sem_q)


def _run(x, lut16):
    mesh = plsc.VectorSubcoreMesh(core_axis_name="c", subcore_axis_name="s")
    return pl.kernel(
        _body,
        out_type=jax.ShapeDtypeStruct((_B, _ITERATIONS + 1, _W), jnp.float32),
        mesh=mesh,
        scratch_types=[
            pltpu.VMEM((_LANES,), jnp.float32),
            pltpu.VMEM((1, _W), jnp.float32),
            pltpu.VMEM((_BUF,), jnp.float32),
            pltpu.VMEM((_BUF,), jnp.float32),
            pltpu.VMEM((8, _W), jnp.float32),
            pltpu.SemaphoreType.DMA,
            pltpu.SemaphoreType.DMA,
            pltpu.SemaphoreType.DMA,
            pltpu.SemaphoreType.DMA,
        ],
    )(x, lut16)


def kernel(input, lookup):
    lut16 = jnp.concatenate([lookup, jnp.zeros((8,), jnp.float32)])
    return _run(input, lut16)
